# baseline (speedup 1.0000x reference)
import functools

import jax
import jax.numpy as jnp
from jax import lax
from jax.experimental import pallas as pl
from jax.experimental.pallas import tpu as pltpu
from jax.experimental.pallas import tpu_sc as plsc

NUM_K = 8192
DIM = 32
PIX = 256
NPIX = 2 * PIX
BETA = 0.25

NCAND = 12
CHUNK = 64
NCHUNKS = NUM_K // CHUNK
NROWS = NPIX * NCAND

NW = 16
ROWS_W = NROWS // NW
IMAX = 2**31 - 1


def _select_kernel(zs_ref, et_ref, cand_ref):
    et = et_ref[...]
    en2 = jnp.sum(et * et, axis=0, keepdims=True)
    kiota = jax.lax.broadcasted_iota(jnp.int32, (PIX, NUM_K), 1)
    for b in range(2):
        zsb = zs_ref[b]
        s = en2 - 2.0 * jax.lax.dot_general(
            zsb, et, (((1,), (0,)), ((), ())),
            preferred_element_type=jnp.float32,
        )
        bi = jax.lax.bitcast_convert_type(s, jnp.int32)
        v = bi ^ jnp.where(bi < 0, jnp.int32(0x7FFFFFFF), jnp.int32(0))
        key = (v & jnp.int32(-8192)) | kiota
        k3 = key.reshape(PIX, NCHUNKS, CHUNK)
        m1 = jnp.min(k3, axis=2, keepdims=True)
        t2 = jnp.where(k3 == m1, IMAX, k3)
        m2 = jnp.min(t2, axis=2, keepdims=True)
        t3 = jnp.where(t2 == m2, IMAX, t2)
        m3 = jnp.min(t3, axis=2, keepdims=True)
        merged = jnp.concatenate([m1, m2, m3], axis=2).reshape(PIX, 3 * NCHUNKS)
        picks = []
        cur = merged
        for _ in range(NCAND):
            g = jnp.min(cur, axis=1, keepdims=True)
            picks.append(g)
            cur = jnp.where(cur == g, IMAX, cur)
        keys12 = jnp.concatenate(picks, axis=1)
        cand_ref[b] = keys12 & jnp.int32(8191)


@functools.cache
def _make_gather_kernel():
    mesh = plsc.VectorSubcoreMesh(
        core_axis_name="c", subcore_axis_name="s", num_cores=1
    )

    @functools.partial(
        pl.kernel,
        mesh=mesh,
        out_type=jax.ShapeDtypeStruct((NROWS, 128), jnp.float32),
        scratch_types=[
            pltpu.VMEM((ROWS_W,), jnp.int32),
            pltpu.VMEM((ROWS_W, 128), jnp.float32),
            pltpu.SemaphoreType.DMA,
        ],
    )
    def _gather_kernel(emb_hbm, idx_hbm, rows_hbm, idx_v, rows_v, sem):
        w = lax.axis_index("s")
        base = w * ROWS_W
        pltpu.sync_copy(idx_hbm.at[pl.ds(base, ROWS_W)], idx_v)
        pltpu.async_copy(emb_hbm.at[idx_v], rows_v, sem).wait()
        pltpu.sync_copy(rows_v, rows_hbm.at[pl.ds(base, ROWS_W)])

    return _gather_kernel


def _rescore_kernel(rt_ref, zs_ref, zn_ref, cand_ref, zqt_ref, idx_ref, loss_ref):
    loss_acc = jnp.zeros((), dtype=jnp.float32)
    for b in range(2):
        zsb = zs_ref[b]
        znb = zn_ref[b]
        cand = cand_ref[b]
        rt = rt_ref[b]
        acc = None
        for c in range(DIM):
            sl = rt[:, c * NCAND:(c + 1) * NCAND]
            zc = zsb[:, c].reshape(PIX, 1)
            d = sl - zc
            sq = d * d
            acc = sq if acc is None else acc + sq
        bd = jnp.full((PIX, 1), jnp.inf, dtype=jnp.float32)
        bk = jnp.full((PIX, 1), NUM_K, dtype=jnp.int32)
        for j in range(NCAND):
            dj = acc[:, j].reshape(PIX, 1)
            kj = cand[:, j].reshape(PIX, 1)
            better = (dj < bd) | ((dj == bd) & (kj < bk))
            bd = jnp.where(better, dj, bd)
            bk = jnp.where(better, kj, bk)
        idx_ref[b] = bk
        wmask = (cand == bk).astype(jnp.float32)
        cols = []
        for c in range(DIM):
            sl = rt[:, c * NCAND:(c + 1) * NCAND]
            cols.append(jnp.sum(sl * wmask, axis=1, keepdims=True))
        zq = jnp.concatenate(cols, axis=1)
        dn = zq - znb
        st = znb + dn
        zqt_ref[b] = st.T
        loss_acc = loss_acc + jnp.sum(dn * dn)
    scale = (1.0 + BETA) / (NPIX * DIM)
    loss_ref[...] = (loss_acc * scale).reshape(1, 1)


def kernel(z, embedding):
    b, c, h, w = z.shape
    zp = jnp.transpose(z, (0, 2, 3, 1))
    flat = zp.reshape(b, h * w * c)
    zs = flat.reshape(b, c, h * w).transpose(0, 2, 1)
    zn = zp.reshape(b, h * w, c)
    et = embedding.T

    cand = pl.pallas_call(
        _select_kernel,
        out_shape=jax.ShapeDtypeStruct((b, h * w, NCAND), jnp.int32),
    )(zs, et)

    emb_pad = jnp.pad(embedding, ((0, 0), (0, 128 - DIM)))
    rows = _make_gather_kernel()(emb_pad, cand.reshape(NROWS))

    rt = rows.reshape(b, h * w, NCAND, 128)[..., :DIM]
    rt = rt.transpose(0, 1, 3, 2).reshape(b, h * w, DIM * NCAND)

    zqt, idx, loss = pl.pallas_call(
        _rescore_kernel,
        out_shape=(
            jax.ShapeDtypeStruct((b, c, h * w), jnp.float32),
            jax.ShapeDtypeStruct((b, h * w, 1), jnp.int32),
            jax.ShapeDtypeStruct((1, 1), jnp.float32),
        ),
    )(rt, zs, zn, cand)

    z_q_out = zqt.reshape(b, c, h, w)
    min_encoding_indices = idx.reshape(b, h, w)
    return (z_q_out, min_encoding_indices, loss.reshape(()))


# speedup vs baseline: 1.8759x; 1.8759x over previous
import functools

import jax
import jax.numpy as jnp
from jax import lax
from jax.experimental import pallas as pl
from jax.experimental.pallas import tpu as pltpu
from jax.experimental.pallas import tpu_sc as plsc

NUM_K = 8192
DIM = 32
PIX = 256
NPIX = 2 * PIX
BETA = 0.25

NCAND = 12
CHUNK = 64
NCHUNKS = NUM_K // CHUNK
NROWS = NPIX * NCAND

NW = 16
ROWS_W = NROWS // NW
IMAX = 2**31 - 1


def _select_kernel(zs_ref, et_ref, cand_ref):
    et = et_ref[...]
    en2 = jnp.sum(et * et, axis=0, keepdims=True)
    kiota = jax.lax.broadcasted_iota(jnp.int32, (PIX, NUM_K), 1)
    for b in range(2):
        zsb = zs_ref[b]
        s = en2 - 2.0 * jax.lax.dot_general(
            zsb, et, (((1,), (0,)), ((), ())),
            preferred_element_type=jnp.float32,
        )
        bi = jax.lax.bitcast_convert_type(s, jnp.int32)
        v = bi ^ jnp.where(bi < 0, jnp.int32(0x7FFFFFFF), jnp.int32(0))
        key = (v & jnp.int32(-8192)) | kiota
        m = key
        sec = None
        width = NUM_K // 2
        while width >= 128:
            am, bm = m[:, :width], m[:, width:2 * width]
            new_m = jnp.minimum(am, bm)
            loser = jnp.maximum(am, bm)
            if sec is None:
                sec = loser
            else:
                a_s, b_s = sec[:, :width], sec[:, width:2 * width]
                sec = jnp.minimum(loser, jnp.minimum(a_s, b_s))
            m = new_m
            width //= 2
        merged = jnp.concatenate([m, sec], axis=1)
        picks = []
        cur = merged
        for _ in range(NCAND):
            g = jnp.min(cur, axis=1, keepdims=True)
            picks.append(g)
            cur = jnp.where(cur == g, IMAX, cur)
        keys12 = jnp.concatenate(picks, axis=1)
        cand_ref[b] = keys12 & jnp.int32(8191)


@functools.cache
def _make_gather_kernel():
    mesh = plsc.VectorSubcoreMesh(
        core_axis_name="c", subcore_axis_name="s", num_cores=1
    )

    @functools.partial(
        pl.kernel,
        mesh=mesh,
        out_type=jax.ShapeDtypeStruct((NROWS, 128), jnp.float32),
        scratch_types=[
            pltpu.VMEM((ROWS_W,), jnp.int32),
            pltpu.VMEM((ROWS_W, 128), jnp.float32),
            pltpu.SemaphoreType.DMA,
        ],
    )
    def _gather_kernel(emb_hbm, idx_hbm, rows_hbm, idx_v, rows_v, sem):
        w = lax.axis_index("s")
        base = w * ROWS_W
        pltpu.sync_copy(idx_hbm.at[pl.ds(base, ROWS_W)], idx_v)
        pltpu.async_copy(emb_hbm.at[idx_v], rows_v, sem).wait()
        pltpu.sync_copy(rows_v, rows_hbm.at[pl.ds(base, ROWS_W)])

    return _gather_kernel


def _rescore_kernel(rt_ref, zs_ref, zn_ref, cand_ref, zqt_ref, idx_ref, loss_ref):
    loss_acc = jnp.zeros((), dtype=jnp.float32)
    for b in range(2):
        zsb = zs_ref[b]
        znb = zn_ref[b]
        cand = cand_ref[b]
        rt = rt_ref[b]
        acc = None
        for c in range(DIM):
            sl = rt[:, c * NCAND:(c + 1) * NCAND]
            zc = zsb[:, c].reshape(PIX, 1)
            d = sl - zc
            sq = d * d
            acc = sq if acc is None else acc + sq
        bd = jnp.full((PIX, 1), jnp.inf, dtype=jnp.float32)
        bk = jnp.full((PIX, 1), NUM_K, dtype=jnp.int32)
        for j in range(NCAND):
            dj = acc[:, j].reshape(PIX, 1)
            kj = cand[:, j].reshape(PIX, 1)
            better = (dj < bd) | ((dj == bd) & (kj < bk))
            bd = jnp.where(better, dj, bd)
            bk = jnp.where(better, kj, bk)
        idx_ref[b] = bk
        wmask = (cand == bk).astype(jnp.float32)
        cols = []
        for c in range(DIM):
            sl = rt[:, c * NCAND:(c + 1) * NCAND]
            cols.append(jnp.sum(sl * wmask, axis=1, keepdims=True))
        zq = jnp.concatenate(cols, axis=1)
        dn = zq - znb
        st = znb + dn
        zqt_ref[b] = st.T
        loss_acc = loss_acc + jnp.sum(dn * dn)
    scale = (1.0 + BETA) / (NPIX * DIM)
    loss_ref[...] = (loss_acc * scale).reshape(1, 1)


def kernel(z, embedding):
    b, c, h, w = z.shape
    zp = jnp.transpose(z, (0, 2, 3, 1))
    flat = zp.reshape(b, h * w * c)
    zs = flat.reshape(b, c, h * w).transpose(0, 2, 1)
    zn = zp.reshape(b, h * w, c)
    et = embedding.T

    cand = pl.pallas_call(
        _select_kernel,
        out_shape=jax.ShapeDtypeStruct((b, h * w, NCAND), jnp.int32),
    )(zs, et)

    emb_pad = jnp.pad(embedding, ((0, 0), (0, 128 - DIM)))
    rows = _make_gather_kernel()(emb_pad, cand.reshape(NROWS))

    rt = rows.reshape(b, h * w, NCAND, 128)[..., :DIM]
    rt = rt.transpose(0, 1, 3, 2).reshape(b, h * w, DIM * NCAND)

    zqt, idx, loss = pl.pallas_call(
        _rescore_kernel,
        out_shape=(
            jax.ShapeDtypeStruct((b, c, h * w), jnp.float32),
            jax.ShapeDtypeStruct((b, h * w, 1), jnp.int32),
            jax.ShapeDtypeStruct((1, 1), jnp.float32),
        ),
    )(rt, zs, zn, cand)

    z_q_out = zqt.reshape(b, c, h, w)
    min_encoding_indices = idx.reshape(b, h, w)
    return (z_q_out, min_encoding_indices, loss.reshape(()))


# speedup vs baseline: 2.2528x; 1.2009x over previous
import functools

import jax
import jax.numpy as jnp
from jax import lax
from jax.experimental import pallas as pl
from jax.experimental.pallas import tpu as pltpu
from jax.experimental.pallas import tpu_sc as plsc

NUM_K = 8192
DIM = 32
PIX = 256
NPIX = 2 * PIX
BETA = 0.25

NCAND = 8
CHUNK = 64
NCHUNKS = NUM_K // CHUNK
NROWS = NPIX * NCAND

NW = 16
ROWS_W = NROWS // NW
IMAX = 2**31 - 1


def _select_kernel(zs_ref, et_ref, cand_ref):
    et = et_ref[...]
    en2 = jnp.sum(et * et, axis=0, keepdims=True)
    kiota = jax.lax.broadcasted_iota(jnp.int32, (PIX, NUM_K), 1)
    for b in range(2):
        zsb = zs_ref[b]
        s = en2 - 2.0 * jax.lax.dot_general(
            zsb, et, (((1,), (0,)), ((), ())),
            preferred_element_type=jnp.float32,
        )
        bi = jax.lax.bitcast_convert_type(s, jnp.int32)
        v = bi ^ jnp.where(bi < 0, jnp.int32(0x7FFFFFFF), jnp.int32(0))
        key = (v & jnp.int32(-8192)) | kiota
        m = key
        sec = None
        width = NUM_K // 2
        while width >= 128:
            am, bm = m[:, :width], m[:, width:2 * width]
            new_m = jnp.minimum(am, bm)
            loser = jnp.maximum(am, bm)
            if sec is None:
                sec = loser
            else:
                a_s, b_s = sec[:, :width], sec[:, width:2 * width]
                sec = jnp.minimum(loser, jnp.minimum(a_s, b_s))
            m = new_m
            width //= 2
        merged = jnp.concatenate([m, sec], axis=1)
        picks = []
        cur = merged
        for _ in range(NCAND):
            g = jnp.min(cur, axis=1, keepdims=True)
            picks.append(g)
            cur = jnp.where(cur == g, IMAX, cur)
        keys12 = jnp.concatenate(picks, axis=1)
        cand_ref[b] = keys12 & jnp.int32(8191)


@functools.cache
def _make_gather_kernel():
    mesh = plsc.VectorSubcoreMesh(
        core_axis_name="c", subcore_axis_name="s", num_cores=1
    )

    @functools.partial(
        pl.kernel,
        mesh=mesh,
        out_type=jax.ShapeDtypeStruct((NROWS, 128), jnp.float32),
        scratch_types=[
            pltpu.VMEM((ROWS_W,), jnp.int32),
            pltpu.VMEM((ROWS_W, 128), jnp.float32),
            pltpu.SemaphoreType.DMA,
        ],
    )
    def _gather_kernel(emb_hbm, idx_hbm, rows_hbm, idx_v, rows_v, sem):
        w = lax.axis_index("s")
        base = w * ROWS_W
        pltpu.sync_copy(idx_hbm.at[pl.ds(base, ROWS_W)], idx_v)
        pltpu.async_copy(emb_hbm.at[idx_v], rows_v, sem).wait()
        pltpu.sync_copy(rows_v, rows_hbm.at[pl.ds(base, ROWS_W)])

    return _gather_kernel


def _rescore_kernel(rt_ref, zs_ref, zn_ref, cand_ref, zqt_ref, idx_ref, loss_ref):
    loss_acc = jnp.zeros((), dtype=jnp.float32)
    for b in range(2):
        zsb = zs_ref[b]
        znb = zn_ref[b]
        cand = cand_ref[b]
        rt = rt_ref[b]
        acc = None
        for c in range(DIM):
            sl = rt[:, c * NCAND:(c + 1) * NCAND]
            zc = zsb[:, c].reshape(PIX, 1)
            d = sl - zc
            sq = d * d
            acc = sq if acc is None else acc + sq
        bd = jnp.full((PIX, 1), jnp.inf, dtype=jnp.float32)
        bk = jnp.full((PIX, 1), NUM_K, dtype=jnp.int32)
        for j in range(NCAND):
            dj = acc[:, j].reshape(PIX, 1)
            kj = cand[:, j].reshape(PIX, 1)
            better = (dj < bd) | ((dj == bd) & (kj < bk))
            bd = jnp.where(better, dj, bd)
            bk = jnp.where(better, kj, bk)
        idx_ref[b] = bk
        wmask = (cand == bk).astype(jnp.float32)
        cols = []
        for c in range(DIM):
            sl = rt[:, c * NCAND:(c + 1) * NCAND]
            cols.append(jnp.sum(sl * wmask, axis=1, keepdims=True))
        zq = jnp.concatenate(cols, axis=1)
        dn = zq - znb
        st = znb + dn
        zqt_ref[b] = st.T
        loss_acc = loss_acc + jnp.sum(dn * dn)
    scale = (1.0 + BETA) / (NPIX * DIM)
    loss_ref[...] = (loss_acc * scale).reshape(1, 1)


def kernel(z, embedding):
    b, c, h, w = z.shape
    zp = jnp.transpose(z, (0, 2, 3, 1))
    flat = zp.reshape(b, h * w * c)
    zs = flat.reshape(b, c, h * w).transpose(0, 2, 1)
    zn = zp.reshape(b, h * w, c)
    et = embedding.T

    cand = pl.pallas_call(
        _select_kernel,
        out_shape=jax.ShapeDtypeStruct((b, h * w, NCAND), jnp.int32),
    )(zs, et)

    emb_pad = jnp.pad(embedding, ((0, 0), (0, 128 - DIM)))
    rows = _make_gather_kernel()(emb_pad, cand.reshape(NROWS))

    rt = rows.reshape(b, h * w, NCAND, 128)[..., :DIM]
    rt = rt.transpose(0, 1, 3, 2).reshape(b, h * w, DIM * NCAND)

    zqt, idx, loss = pl.pallas_call(
        _rescore_kernel,
        out_shape=(
            jax.ShapeDtypeStruct((b, c, h * w), jnp.float32),
            jax.ShapeDtypeStruct((b, h * w, 1), jnp.int32),
            jax.ShapeDtypeStruct((1, 1), jnp.float32),
        ),
    )(rt, zs, zn, cand)

    z_q_out = zqt.reshape(b, c, h, w)
    min_encoding_indices = idx.reshape(b, h, w)
    return (z_q_out, min_encoding_indices, loss.reshape(()))


# speedup vs baseline: 3.1042x; 1.3779x over previous
import functools

import jax
import jax.numpy as jnp
from jax import lax
from jax.experimental import pallas as pl
from jax.experimental.pallas import tpu as pltpu
from jax.experimental.pallas import tpu_sc as plsc

NUM_K = 8192
DIM = 32
PIX = 256
NPIX = 2 * PIX
BETA = 0.25

NCAND = 8
CHUNK = 64
NCHUNKS = NUM_K // CHUNK
NROWS = NPIX * NCAND

NW = 16
ROWS_W = NROWS // NW
IMAX = 2**31 - 1


def _select_kernel(zs_ref, et_ref, cand_ref):
    et = et_ref[...]
    en2 = jnp.sum(et * et, axis=0, keepdims=True)
    kiota = jax.lax.broadcasted_iota(jnp.int32, (PIX, NUM_K), 1)
    for b in range(2):
        zsb = zs_ref[b]
        s = en2 - 2.0 * jax.lax.dot_general(
            zsb, et, (((1,), (0,)), ((), ())),
            preferred_element_type=jnp.float32,
        )
        bi = jax.lax.bitcast_convert_type(s, jnp.int32)
        v = bi ^ jnp.where(bi < 0, jnp.int32(0x7FFFFFFF), jnp.int32(0))
        key = (v & jnp.int32(-8192)) | kiota
        m = key
        sec = None
        width = NUM_K // 2
        while width >= 128:
            am, bm = m[:, :width], m[:, width:2 * width]
            new_m = jnp.minimum(am, bm)
            loser = jnp.maximum(am, bm)
            if sec is None:
                sec = loser
            else:
                a_s, b_s = sec[:, :width], sec[:, width:2 * width]
                sec = jnp.minimum(loser, jnp.minimum(a_s, b_s))
            m = new_m
            width //= 2
        merged = jnp.concatenate([m, sec], axis=1)
        picks = []
        cur = merged
        for _ in range(NCAND):
            g = jnp.min(cur, axis=1, keepdims=True)
            picks.append(g)
            cur = jnp.where(cur == g, IMAX, cur)
        keys12 = jnp.concatenate(picks, axis=1)
        cand_ref[b] = keys12 & jnp.int32(8191)


@functools.cache
def _make_gather_kernel():
    mesh = plsc.VectorSubcoreMesh(
        core_axis_name="c", subcore_axis_name="s", num_cores=1
    )

    @functools.partial(
        pl.kernel,
        mesh=mesh,
        out_type=jax.ShapeDtypeStruct((NROWS, 128), jnp.float32),
        scratch_types=[
            pltpu.VMEM((ROWS_W,), jnp.int32),
            pltpu.VMEM((ROWS_W, 128), jnp.float32),
            pltpu.SemaphoreType.DMA,
        ],
    )
    def _gather_kernel(emb_hbm, idx_hbm, rows_hbm, idx_v, rows_v, sem):
        w = lax.axis_index("s")
        base = w * ROWS_W
        pltpu.sync_copy(idx_hbm.at[pl.ds(base, ROWS_W)], idx_v)
        pltpu.async_copy(emb_hbm.at[idx_v], rows_v, sem).wait()
        pltpu.sync_copy(rows_v, rows_hbm.at[pl.ds(base, ROWS_W)])

    return _gather_kernel


def _rescore_kernel(rt_ref, zst_ref, znt_ref, candl_ref, zqt_ref, idx_ref, loss_ref):
    loss_acc = jnp.zeros((), dtype=jnp.float32)
    for b in range(2):
        rt = rt_ref[b]
        zst = zst_ref[b]
        znt = znt_ref[b]
        candl = candl_ref[b]
        acc = None
        for c in range(DIM):
            zrow = zst[c:c + 1, :]
            zfull = jnp.concatenate([zrow] * NCAND, axis=1)
            d = rt[c:c + 1, :] - zfull
            sq = d * d
            acc = sq if acc is None else acc + sq
        dcur, kcur = acc, candl
        width = NCAND * PIX // 2
        while width >= PIX:
            ad, bd_ = dcur[:, :width], dcur[:, width:2 * width]
            ak, bk_ = kcur[:, :width], kcur[:, width:2 * width]
            better = (bd_ < ad) | ((bd_ == ad) & (bk_ < ak))
            dcur = jnp.where(better, bd_, ad)
            kcur = jnp.where(better, bk_, ak)
            width //= 2
        idx_ref[b] = kcur
        kwin_full = jnp.concatenate([kcur] * NCAND, axis=1)
        wmask = (candl == kwin_full).astype(jnp.float32)
        rows_c = []
        for c in range(DIM):
            sel = rt[c:c + 1, :] * wmask
            w2 = NCAND * PIX // 2
            while w2 >= PIX:
                sel = sel[:, :w2] + sel[:, w2:2 * w2]
                w2 //= 2
            rows_c.append(sel)
        zq = jnp.concatenate(rows_c, axis=0)
        dn = zq - znt
        st = znt + dn
        zqt_ref[b] = st
        loss_acc = loss_acc + jnp.sum(dn * dn)
    scale = (1.0 + BETA) / (NPIX * DIM)
    loss_ref[...] = (loss_acc * scale).reshape(1, 1)


def kernel(z, embedding):
    b, c, h, w = z.shape
    zp = jnp.transpose(z, (0, 2, 3, 1))
    flat = zp.reshape(b, h * w * c)
    zs = flat.reshape(b, c, h * w).transpose(0, 2, 1)
    zn = zp.reshape(b, h * w, c)
    et = embedding.T

    cand = pl.pallas_call(
        _select_kernel,
        out_shape=jax.ShapeDtypeStruct((b, h * w, NCAND), jnp.int32),
    )(zs, et)

    emb_pad = jnp.pad(embedding, ((0, 0), (0, 128 - DIM)))
    rows = _make_gather_kernel()(emb_pad, cand.reshape(NROWS))

    rt = rows.reshape(b, h * w, NCAND, 128)[..., :DIM]
    rt = rt.transpose(0, 3, 2, 1).reshape(b, c, NCAND * h * w)
    candl = cand.transpose(0, 2, 1).reshape(b, 1, NCAND * h * w)
    zst = zs.transpose(0, 2, 1)
    znt = zn.transpose(0, 2, 1)

    zqt, idx, loss = pl.pallas_call(
        _rescore_kernel,
        out_shape=(
            jax.ShapeDtypeStruct((b, c, h * w), jnp.float32),
            jax.ShapeDtypeStruct((b, 1, h * w), jnp.int32),
            jax.ShapeDtypeStruct((1, 1), jnp.float32),
        ),
    )(rt, zst, znt, candl)

    z_q_out = zqt.reshape(b, c, h, w)
    min_encoding_indices = idx.reshape(b, h, w)
    return (z_q_out, min_encoding_indices, loss.reshape(()))
